# baseline (speedup 1.0000x reference)
import functools
import math

import jax
import jax.numpy as jnp
from jax import lax
from jax.experimental import pallas as pl
from jax.experimental.pallas import tpu as pltpu
from jax.experimental.pallas import tpu_sc as plsc

_VOCAB = 1000000
_D = 64
_B = 4096
_S = 200

_NC, _NS = 2, 16
_NW = _NC * _NS
_SB = _S * _B
_RPW = _SB // _NW
_C = 256
_NCHUNK = _RPW // _C
_G = 128
_NG = _C // _G


def _sinusoidal_pe() -> jnp.ndarray:
    position = jnp.arange(_S, dtype=jnp.float32)[:, None]
    div_term = jnp.exp(
        jnp.arange(0, _D, 2, dtype=jnp.float32) * (-math.log(10000.0) / _D))
    pe = jnp.zeros((_S, _D), dtype=jnp.float32)
    pe = pe.at[:, 0::2].set(jnp.sin(position * div_term))
    pe = pe.at[:, 1::2].set(jnp.cos(position * div_term))
    return pe


@functools.partial(
    pl.kernel,
    out_type=jax.ShapeDtypeStruct((_SB, 2 * _D), jnp.float32),
    mesh=plsc.VectorSubcoreMesh(core_axis_name="c", subcore_axis_name="s"),
    scratch_types=[
        pltpu.VMEM((2, _C), jnp.int32),
        pltpu.VMEM((2, _C, 2 * _D), jnp.float32),
        pltpu.SemaphoreType.DMA,
        pltpu.SemaphoreType.DMA,
        pltpu.SemaphoreType.DMA,
        pltpu.SemaphoreType.DMA,
    ],
)
def _gather_kernel(q_hbm, table_hbm, out_hbm, q_v, rows_v, g0, g1, o0, o1):
    wid = lax.axis_index("s") * _NC + lax.axis_index("c")
    base = wid * _RPW
    gsems = (g0, g1)
    osems = (o0, o1)

    def stage_and_fire(g, buf):
        row_base = base + g * _C
        pltpu.sync_copy(q_hbm.at[pl.ds(row_base, _C)], q_v.at[buf])
        for k in range(_NG):
            pltpu.async_copy(
                table_hbm.at[q_v.at[buf, pl.ds(k * _G, _G)]],
                rows_v.at[buf, pl.ds(k * _G, _G)],
                gsems[buf],
            )

    def drain_gathers(buf):
        for k in range(_NG):
            pltpu.make_async_copy(
                table_hbm.at[q_v.at[buf, pl.ds(k * _G, _G)]],
                rows_v.at[buf, pl.ds(k * _G, _G)],
                gsems[buf],
            ).wait()

    def out_copy(g, buf):
        row_base = base + g * _C
        return pltpu.make_async_copy(
            rows_v.at[buf],
            out_hbm.at[pl.ds(row_base, _C)],
            osems[buf],
        )

    stage_and_fire(0, 0)

    def pair_body(g2, _):
        for b in range(2):
            g = g2 * 2 + b

            @pl.when(g >= 1)
            def _():
                out_copy(g - 1, 1 - b).wait()

            @pl.when(g + 1 < _NCHUNK)
            def _():
                stage_and_fire(g + 1, 1 - b)

            drain_gathers(b)
            out_copy(g, b).start()
        return 0

    lax.fori_loop(0, _NCHUNK // 2, pair_body, 0)
    out_copy(_NCHUNK - 1, 1).wait()


def _finish_block(pairs_ref, par_ref, pe_ref, out_ref):
    y = pairs_ref[0]
    lo = y[:, :_D]
    hi = y[:, _D:]
    m = par_ref[0] > 0
    out_ref[0] = jnp.where(m, hi, lo) + pe_ref[0, 0][None, :]


def kernel(x, table):
    idx = jnp.transpose(x).astype(jnp.int32)
    q = (idx >> 1).reshape(_SB)
    par = idx & 1
    table_pairs = table.reshape(_VOCAB // 2, 2 * _D)
    pe = _sinusoidal_pe()

    pairs = _gather_kernel(q, table_pairs)
    pairs = pairs.reshape(_S, _B, 2 * _D)

    return pl.pallas_call(
        _finish_block,
        grid=(_S,),
        in_specs=[
            pl.BlockSpec((1, _B, 2 * _D), lambda s: (s, 0, 0)),
            pl.BlockSpec((1, _B, 1), lambda s: (s, 0, 0)),
            pl.BlockSpec((1, 1, _D), lambda s: (s, 0, 0)),
        ],
        out_specs=pl.BlockSpec((1, _B, _D), lambda s: (s, 0, 0)),
        out_shape=jax.ShapeDtypeStruct((_S, _B, _D), jnp.float32),
    )(pairs, par.reshape(_S, _B, 1), pe.reshape(_S, 1, _D))


# speedup vs baseline: 1.4120x; 1.4120x over previous
import functools
import math

import jax
import jax.numpy as jnp
from jax import lax
from jax.experimental import pallas as pl
from jax.experimental.pallas import tpu as pltpu
from jax.experimental.pallas import tpu_sc as plsc

_VOCAB = 1000000
_D = 64
_B = 4096
_S = 200

_NC, _NS, _L = 2, 16, 16
_NW = _NC * _NS
_SB = _S * _B
_RPW = _SB // _NW
_C = 512
_NCHUNK = _RPW // _C
_G = 128
_NG = _C // _G


def _sinusoidal_pe() -> jnp.ndarray:
    position = jnp.arange(_S, dtype=jnp.float32)[:, None]
    div_term = jnp.exp(
        jnp.arange(0, _D, 2, dtype=jnp.float32) * (-math.log(10000.0) / _D))
    pe = jnp.zeros((_S, _D), jnp.float32)
    pe = pe.at[:, 0::2].set(jnp.sin(position * div_term))
    pe = pe.at[:, 1::2].set(jnp.cos(position * div_term))
    return pe


@functools.partial(
    pl.kernel,
    out_type=jax.ShapeDtypeStruct((_S, _B, _D), jnp.float32),
    mesh=plsc.VectorSubcoreMesh(core_axis_name="c", subcore_axis_name="s"),
    compiler_params=pltpu.CompilerParams(use_tc_tiling_on_sc=False),
    scratch_types=[
        pltpu.VMEM((2, _C), jnp.int32),
        pltpu.VMEM((2, _C, _D), jnp.float32),
        pltpu.VMEM((_S, _D), jnp.float32),
        pltpu.SemaphoreType.DMA,
        pltpu.SemaphoreType.DMA,
        pltpu.SemaphoreType.DMA,
        pltpu.SemaphoreType.DMA,
    ],
)
def _emb_kernel(idx_hbm, table_hbm, pe_hbm, out_hbm,
                idx_v, rows_v, pe_v, g0, g1, o0, o1):
    wid = lax.axis_index("s") * _NC + lax.axis_index("c")
    base = wid * _RPW
    gsems = (g0, g1)
    osems = (o0, o1)
    pltpu.sync_copy(pe_hbm, pe_v)

    def stage_and_fire(g, buf):
        row_base = base + g * _C
        pltpu.sync_copy(idx_hbm.at[pl.ds(row_base, _C)], idx_v.at[buf])
        for k in range(_NG):
            pltpu.async_copy(
                table_hbm.at[idx_v.at[buf, pl.ds(k * _G, _G)]],
                rows_v.at[buf, pl.ds(k * _G, _G)],
                gsems[buf],
            )

    def drain_gathers(buf):
        for k in range(_NG):
            pltpu.make_async_copy(
                table_hbm.at[idx_v.at[buf, pl.ds(k * _G, _G)]],
                rows_v.at[buf, pl.ds(k * _G, _G)],
                gsems[buf],
            ).wait()

    def out_copy(g, buf):
        row_base = base + g * _C
        s = row_base // _B
        b0 = row_base % _B
        return pltpu.make_async_copy(
            rows_v.at[buf],
            out_hbm.at[s, pl.ds(b0, _C)],
            osems[buf],
        )

    def add_pe(g, buf):
        s = (base + g * _C) // _B
        pes = tuple(pe_v[s, pl.ds(j * _L, _L)] for j in range(_D // _L))

        def row_body(i, ps):
            for j in range(_D // _L):
                sl = pl.ds(j * _L, _L)
                rows_v[buf, i, sl] = rows_v[buf, i, sl] + ps[j]
            return ps

        lax.fori_loop(0, _C, row_body, pes)

    stage_and_fire(0, 0)

    def pair_body(g2, _):
        for b in range(2):
            g = g2 * 2 + b

            @pl.when(g >= 1)
            def _():
                out_copy(g - 1, 1 - b).wait()

            @pl.when(g + 1 < _NCHUNK)
            def _():
                stage_and_fire(g + 1, 1 - b)

            drain_gathers(b)
            add_pe(g, b)
            out_copy(g, b).start()
        return 0

    lax.fori_loop(0, _NCHUNK // 2, pair_body, 0)
    out_copy(_NCHUNK - 1, 1).wait()


def kernel(x, table):
    idx = jnp.transpose(x).reshape(_SB).astype(jnp.int32)
    pe = _sinusoidal_pe()
    return _emb_kernel(idx, table, pe)
